# baseline (speedup 1.0000x reference)
import functools

import jax
import jax.numpy as jnp
from jax import lax
from jax.experimental import pallas as pl
from jax.experimental.pallas import tpu as pltpu
from jax.experimental.pallas import tpu_sc as plsc

_L = 200
_D = 64

_info = plsc.get_sparse_core_info()
_NC = _info.num_cores
_NS = _info.num_subcores
_NW = _NC * _NS


def _sc_pool(x, table):
    B = x.shape[1]
    bpw = B // _NW
    bpc = bpw * _NS

    mesh = plsc.VectorSubcoreMesh(core_axis_name="c", subcore_axis_name="s")

    @functools.partial(
        pl.kernel,
        mesh=mesh,
        out_type=jax.ShapeDtypeStruct((B, _D), jnp.float32),
        scratch_types=[
            pltpu.VMEM((_L, bpw), jnp.int32),
            pltpu.VMEM((bpw, _D), jnp.float32),
            pltpu.VMEM((bpw,), jnp.int32),
            pltpu.VMEM_SHARED((bpc, _D), jnp.float32),
        ],
        compiler_params=pltpu.CompilerParams(use_tc_tiling_on_sc=False),
    )
    def pool(x_hbm, table_hbm, out_hbm, idx_v, rows_v, slot_v, acc_sh):
        cid = lax.axis_index("c")
        sid = lax.axis_index("s")
        base = (sid * _NC + cid) * bpw
        sbase = sid * bpw

        for k in range(bpw // 16):
            slot_v[pl.ds(k * 16, 16)] = sbase + k * 16 + lax.iota(jnp.int32, 16)

        pltpu.sync_copy(x_hbm.at[:, pl.ds(base, bpw)], idx_v)

        pltpu.sync_copy(table_hbm.at[idx_v.at[0]], rows_v)
        pltpu.sync_copy(rows_v, acc_sh.at[slot_v])

        def step(l, carry):
            pltpu.sync_copy(table_hbm.at[idx_v.at[l]], rows_v)
            pltpu.sync_copy(rows_v, acc_sh.at[slot_v], add=True)
            return carry

        lax.fori_loop(1, _L, step, 0)

        pltpu.sync_copy(acc_sh.at[pl.ds(sbase, bpw)], out_hbm.at[pl.ds(base, bpw)])

    return pool(x, table)


def _mlp(sums, W1, b1, w2, b2):
    B = sums.shape[0]

    def body(s_ref, w1_ref, b1_ref, w2_ref, b2_ref, o_ref):
        m = s_ref[...] * (1.0 / _L)
        h = jnp.dot(m, w1_ref[...], preferred_element_type=jnp.float32)
        h = jnp.maximum(h + b1_ref[...][None, :], 0.0)
        z = jnp.sum(h * w2_ref[...][None, :], axis=-1) + b2_ref[0, 0]
        o_ref[...] = (1.0 / (1.0 + jnp.exp(-z)))[:, None]

    return pl.pallas_call(
        body,
        out_shape=jax.ShapeDtypeStruct((B, 1), jnp.float32),
    )(sums, W1, b1, w2, b2)


def kernel(x, table, W1, b1, W2, b2):
    x = x.astype(jnp.int32)
    sums = _sc_pool(x, table)
    out = _mlp(sums, W1, b1, W2.reshape(_D), b2.reshape(1, 1))
    return out.reshape(x.shape[1])


# speedup vs baseline: 1.2334x; 1.2334x over previous
import functools

import jax
import jax.numpy as jnp
from jax import lax
from jax.experimental import pallas as pl
from jax.experimental.pallas import tpu as pltpu
from jax.experimental.pallas import tpu_sc as plsc

_L = 200
_D = 64

_info = plsc.get_sparse_core_info()
_NC = _info.num_cores
_NS = _info.num_subcores
_NW = _NC * _NS


def _sc_pool(x, table):
    B = x.shape[1]
    bpw = B // _NW
    bpc = bpw * _NS

    mesh = plsc.VectorSubcoreMesh(core_axis_name="c", subcore_axis_name="s")
    nbuf = 8
    nch = _L // nbuf

    @functools.partial(
        pl.kernel,
        mesh=mesh,
        out_type=jax.ShapeDtypeStruct((B, _D), jnp.float32),
        scratch_types=[
            pltpu.VMEM((_L, bpw), jnp.int32),
            pltpu.VMEM((nbuf, bpw, _D), jnp.float32),
            pltpu.VMEM((bpw,), jnp.int32),
            pltpu.VMEM_SHARED((bpc, _D), jnp.float32),
            pltpu.SemaphoreType.DMA((nbuf,)),
            pltpu.SemaphoreType.DMA((nbuf,)),
        ],
        compiler_params=pltpu.CompilerParams(use_tc_tiling_on_sc=False),
    )
    def pool(x_hbm, table_hbm, out_hbm, idx_v, bufs, slot_v, acc_sh, gsem, ssem):
        cid = lax.axis_index("c")
        sid = lax.axis_index("s")
        base = (sid * _NC + cid) * bpw
        sbase = sid * bpw

        zvec = jnp.zeros((16,), jnp.float32)
        for k in range(bpw // 16):
            slot_v[pl.ds(k * 16, 16)] = sbase + k * 16 + lax.iota(jnp.int32, 16)

        def zero_row(i, carry):
            for d in range(_D // 16):
                bufs[0, i, pl.ds(d * 16, 16)] = zvec
            return carry
        lax.fori_loop(0, bpw, zero_row, 0)
        pltpu.sync_copy(bufs.at[0], acc_sh.at[pl.ds(sbase, bpw)])

        pltpu.sync_copy(x_hbm.at[:, pl.ds(base, bpw)], idx_v)

        def gather(l, b):
            return pltpu.async_copy(
                table_hbm.at[idx_v.at[l]], bufs.at[b], gsem.at[b])

        def scatter_add(b):
            return pltpu.async_copy(
                bufs.at[b], acc_sh.at[slot_v], ssem.at[b], add=True)

        for b in range(nbuf):
            gather(b, b)

        def chunk(c, carry):
            l0 = c * nbuf
            for b in range(nbuf):
                pltpu.make_async_copy(
                    table_hbm.at[idx_v.at[l0 + b]], bufs.at[b], gsem.at[b]).wait()
                scatter_add(b)
                pltpu.make_async_copy(
                    bufs.at[b], acc_sh.at[slot_v], ssem.at[b]).wait()
                gather(l0 + b + nbuf, b)
            return carry

        lax.fori_loop(0, nch - 1, chunk, 0)

        l0 = (nch - 1) * nbuf
        for b in range(nbuf):
            pltpu.make_async_copy(
                table_hbm.at[idx_v.at[l0 + b]], bufs.at[b], gsem.at[b]).wait()
            scatter_add(b)
        for b in range(nbuf):
            pltpu.make_async_copy(
                bufs.at[b], acc_sh.at[slot_v], ssem.at[b]).wait()

        pltpu.sync_copy(acc_sh.at[pl.ds(sbase, bpw)], out_hbm.at[pl.ds(base, bpw)])

    return pool(x, table)


def _mlp(sums, W1, b1, w2, b2):
    B = sums.shape[0]

    def body(s_ref, w1_ref, b1_ref, w2_ref, b2_ref, o_ref):
        m = s_ref[...] * (1.0 / _L)
        h = jnp.dot(m, w1_ref[...], preferred_element_type=jnp.float32)
        h = jnp.maximum(h + b1_ref[...][None, :], 0.0)
        z = jnp.sum(h * w2_ref[...][None, :], axis=-1) + b2_ref[0, 0]
        o_ref[...] = (1.0 / (1.0 + jnp.exp(-z)))[:, None]

    return pl.pallas_call(
        body,
        out_shape=jax.ShapeDtypeStruct((B, 1), jnp.float32),
    )(sums, W1, b1, w2, b2)


def kernel(x, table, W1, b1, W2, b2):
    x = x.astype(jnp.int32)
    sums = _sc_pool(x, table)
    out = _mlp(sums, W1, b1, W2.reshape(_D), b2.reshape(1, 1))
    return out.reshape(x.shape[1])
